# baseline (speedup 1.0000x reference)
import functools

import jax
import jax.numpy as jnp
from jax import lax
from jax.experimental import pallas as pl
from jax.experimental.pallas import tpu as pltpu
from jax.experimental.pallas import tpu_sc as plsc

jax.config.update("jax_enable_x64", True)

SINK_SIZE = 4
WINDOW_SIZE = 8192
MAX_CONTEXT = SINK_SIZE + WINDOW_SIZE * 2
SEQ_LEN = 2048

NUM_WORKERS = 32
CP_PAD = 16896
CP_CHUNK = CP_PAD // NUM_WORKERS
IDX_CHUNK = SEQ_LEN // NUM_WORKERS
LANES = 16


def _sc_body(scal_hbm, cp_hbm, idx_hbm, out_hbm, s_ref, cp_ref, idx_ref, out_ref):
    pltpu.sync_copy(scal_hbm, s_ref)
    return
    wid = lax.axis_index("s") * 2 + lax.axis_index("c")
    base = wid * CP_CHUNK

    pltpu.sync_copy(scal_hbm, s_ref)
    pltpu.sync_copy(cp_hbm.at[pl.ds(base, CP_CHUNK)], cp_ref)

    sp = s_ref[pl.ds(0, LANES)]
    se = s_ref[pl.ds(LANES, LANES)]
    hi = se + SEQ_LEN
    lane = jnp.arange(LANES, dtype=jnp.int32)

    for i in range(CP_CHUNK // LANES):
        j = lane + (base + i * LANES)
        old = cp_ref[pl.ds(i * LANES, LANES)]
        c1 = j + 2 * WINDOW_SIZE
        c1_ok = (j >= SINK_SIZE) & (c1 >= se) & (c1 < hi)
        c0_ok = (j >= se) & (j < hi)
        keep = (j < SINK_SIZE) | (j < sp)
        merged = jnp.where(keep, old, jnp.full_like(j, -1))
        out_ref[pl.ds(i * LANES, LANES)] = jnp.where(
            c1_ok, c1, jnp.where(c0_ok, j, merged)
        )

    ib = wid * IDX_CHUNK
    for i in range(IDX_CHUNK // LANES):
        orig = lane + ib + i * LANES + se
        win = SINK_SIZE + jnp.bitwise_and(
            jnp.maximum(orig - SINK_SIZE, 0), 2 * WINDOW_SIZE - 1
        )
        idx_ref[pl.ds(i * LANES, LANES)] = jnp.where(
            orig < SINK_SIZE, jnp.minimum(orig, SINK_SIZE), win
        )

    pltpu.sync_copy(idx_ref, idx_hbm.at[pl.ds(ib, IDX_CHUNK)])
    pltpu.sync_copy(out_ref, out_hbm.at[pl.ds(base, CP_CHUNK)])


@functools.partial(jax.jit, static_argnames=())
def _run_sc(scal, cp_pad):
    mesh = plsc.VectorSubcoreMesh(core_axis_name="c", subcore_axis_name="s")
    return pl.kernel(
        _sc_body,
        mesh=mesh,
        out_type=[
            jax.ShapeDtypeStruct((SEQ_LEN,), jnp.int32),
            jax.ShapeDtypeStruct((CP_PAD,), jnp.int32),
        ],
        scratch_types=[
            pltpu.VMEM((2,), jnp.int32),
            pltpu.VMEM((CP_CHUNK,), jnp.int32),
            pltpu.VMEM((IDX_CHUNK,), jnp.int32),
            pltpu.VMEM((CP_CHUNK,), jnp.int32),
        ],
    )(scal, cp_pad)


def kernel(input_pos, seq_len, cache_positions):
    scal = lax.bitcast_convert_type(input_pos, jnp.int32).reshape(2)
    cp_view = lax.bitcast_convert_type(cache_positions, jnp.int32)
    idx32, out32 = _run_sc(scal, cp_view)
    return idx32, out32[:MAX_CONTEXT]


# speedup vs baseline: 1.1993x; 1.1993x over previous
import functools

import jax
import jax.numpy as jnp
from jax import lax
from jax.experimental import pallas as pl
from jax.experimental.pallas import tpu as pltpu
from jax.experimental.pallas import tpu_sc as plsc

jax.config.update("jax_enable_x64", True)

SINK_SIZE = 4
WINDOW_SIZE = 8192
MAX_CONTEXT = SINK_SIZE + WINDOW_SIZE * 2
SEQ_LEN = 2048

NUM_WORKERS = 32
CP_PAD = 16896
CP_CHUNK = CP_PAD // NUM_WORKERS
IDX_CHUNK = SEQ_LEN // NUM_WORKERS
LANES = 16


def _sc_body(
    scal_hbm, cp_hbm, idx_hbm, out_hbm, s_ref, cp_ref, idx_ref, out_ref,
    sem_s, sem_c, sem_i,
):
    wid = lax.axis_index("s") * 2 + lax.axis_index("c")
    base = wid * CP_CHUNK

    h_s = pltpu.async_copy(scal_hbm, s_ref, sem_s)
    h_c = pltpu.async_copy(cp_hbm.at[pl.ds(base, CP_CHUNK)], cp_ref, sem_c)
    h_s.wait()

    sp = s_ref[pl.ds(0, LANES)]
    se = s_ref[pl.ds(LANES, LANES)]
    hi = se + SEQ_LEN
    lane = jnp.arange(LANES, dtype=jnp.int32)

    ib = wid * IDX_CHUNK
    for i in range(IDX_CHUNK // LANES):
        orig = lane + ib + i * LANES + se
        win = SINK_SIZE + jnp.bitwise_and(
            jnp.maximum(orig - SINK_SIZE, 0), 2 * WINDOW_SIZE - 1
        )
        idx_ref[pl.ds(i * LANES, LANES)] = jnp.where(
            orig < SINK_SIZE, jnp.minimum(orig, SINK_SIZE), win
        )
    h_i = pltpu.async_copy(idx_ref, idx_hbm.at[pl.ds(ib, IDX_CHUNK)], sem_i)

    h_c.wait()
    for i in range(CP_CHUNK // LANES):
        j = lane + (base + i * LANES)
        old = cp_ref[pl.ds(i * LANES, LANES)]
        c1 = j + 2 * WINDOW_SIZE
        c1_ok = (j >= SINK_SIZE) & (c1 >= se) & (c1 < hi)
        c0_ok = (j >= se) & (j < hi)
        keep = (j < SINK_SIZE) | (j < sp)
        merged = jnp.where(keep, old, jnp.full_like(j, -1))
        out_ref[pl.ds(i * LANES, LANES)] = jnp.where(
            c1_ok, c1, jnp.where(c0_ok, j, merged)
        )

    pltpu.sync_copy(out_ref, out_hbm.at[pl.ds(base, CP_CHUNK)])
    h_i.wait()


@functools.partial(jax.jit, static_argnames=())
def _run_sc(scal, cp_pad):
    mesh = plsc.VectorSubcoreMesh(core_axis_name="c", subcore_axis_name="s")
    return pl.kernel(
        _sc_body,
        mesh=mesh,
        out_type=[
            jax.ShapeDtypeStruct((SEQ_LEN,), jnp.int32),
            jax.ShapeDtypeStruct((CP_PAD,), jnp.int32),
        ],
        scratch_types=[
            pltpu.VMEM((2 * LANES,), jnp.int32),
            pltpu.VMEM((CP_CHUNK,), jnp.int32),
            pltpu.VMEM((IDX_CHUNK,), jnp.int32),
            pltpu.VMEM((CP_CHUNK,), jnp.int32),
            pltpu.SemaphoreType.DMA,
            pltpu.SemaphoreType.DMA,
            pltpu.SemaphoreType.DMA,
        ],
    )(scal, cp_pad)


def kernel(input_pos, seq_len, cache_positions):
    sp = input_pos[0]
    se = sp + jnp.asarray(seq_len, sp.dtype) - SEQ_LEN
    scal = jnp.concatenate(
        [
            jnp.full((LANES,), sp.astype(jnp.int32)),
            jnp.full((LANES,), se.astype(jnp.int32)),
        ]
    )
    cp_pad = jnp.concatenate(
        [
            cache_positions.astype(jnp.int32),
            jnp.zeros((CP_PAD - MAX_CONTEXT,), jnp.int32),
        ]
    )
    idx32, out32 = _run_sc(scal, cp_pad)
    return idx32.astype(jnp.int64), out32[:MAX_CONTEXT].astype(jnp.int64)
